# Initial kernel scaffold; baseline (speedup 1.0000x reference)
import functools

import jax
import jax.numpy as jnp
from jax import lax
from jax.experimental import pallas as pl
from jax.experimental.pallas import tpu as pltpu
from jax.experimental.pallas import tpu_sc as plsc

N = 10000
E = 320000
D_IN = 128
HID = 128
LAT = 64
OUT = 32

NC = 2
NS = 16
NW = NC * NS
CHUNK = 128
CPT = 80
NCHUNKS = NW * CPT
EP = NCHUNKS * CHUNK
NPAD = 10240
RPT = NPAD // NS
ZROWS = 128

_mesh = plsc.VectorSubcoreMesh(
    core_axis_name="c", subcore_axis_name="s", num_cores=NC, num_subcores=NS)


def _zero_fill(buf, rows, width):
    def row(r, _):
        for k in range(width // 16):
            buf[r, pl.ds(k * 16, 16)] = jnp.zeros((16,), jnp.float32)
        return 0
    lax.fori_loop(0, rows, row, 0, unroll=False)


def _make_prop(width):
    @functools.partial(
        pl.kernel,
        mesh=_mesh,
        out_type=jax.ShapeDtypeStruct((NC, NPAD, width), jnp.float32),
        scratch_types=[
            pltpu.VMEM((CPT, CHUNK), jnp.int32),
            pltpu.VMEM((CPT, CHUNK), jnp.int32),
            pltpu.VMEM((CHUNK, width), jnp.float32),
            pltpu.VMEM((CHUNK, width), jnp.float32),
            pltpu.VMEM((ZROWS, width), jnp.float32),
            pltpu.VMEM_SHARED((NPAD, width), jnp.float32),
            pltpu.SemaphoreType.DMA,
            pltpu.SemaphoreType.DMA,
        ],
    )
    def prop(src_hbm, dst_hbm, u_hbm, out_hbm, sidx, didx, rows_a, rows_b,
             zbuf, acc, sem_a, sem_b):
        c = lax.axis_index("c")
        s = lax.axis_index("s")
        wid = s * NC + c

        _zero_fill(zbuf, ZROWS, width)
        for t in range(RPT // ZROWS):
            pltpu.sync_copy(zbuf, acc.at[pl.ds(s * RPT + t * ZROWS, ZROWS)])

        pltpu.sync_copy(src_hbm.at[pl.ds(wid * CPT, CPT)], sidx)
        pltpu.sync_copy(dst_hbm.at[pl.ds(wid * CPT, CPT)], didx)
        plsc.subcore_barrier()

        pltpu.async_copy(u_hbm.at[sidx.at[0]], rows_a, sem_a)

        def body(i, _):
            j0 = 2 * i
            pltpu.async_copy(u_hbm.at[sidx.at[j0 + 1]], rows_b, sem_b)
            pltpu.make_async_copy(u_hbm.at[sidx.at[j0]], rows_a, sem_a).wait()
            pltpu.sync_copy(rows_a, acc.at[didx.at[j0]], add=True)

            @pl.when(i < CPT // 2 - 1)
            def _():
                pltpu.async_copy(u_hbm.at[sidx.at[j0 + 2]], rows_a, sem_a)

            pltpu.make_async_copy(u_hbm.at[sidx.at[j0 + 1]], rows_b, sem_b).wait()
            pltpu.sync_copy(rows_b, acc.at[didx.at[j0 + 1]], add=True)
            return 0

        lax.fori_loop(0, CPT // 2, body, 0, unroll=False)
        plsc.subcore_barrier()

        pltpu.sync_copy(acc.at[pl.ds(s * RPT, RPT)],
                        out_hbm.at[c, pl.ds(s * RPT, RPT)])

    return prop


DEG_W = 16


@functools.partial(
    pl.kernel,
    mesh=_mesh,
    out_type=jax.ShapeDtypeStruct((NC, NPAD, DEG_W), jnp.float32),
    scratch_types=[
        pltpu.VMEM((CPT, CHUNK), jnp.int32),
        pltpu.VMEM((CHUNK, DEG_W), jnp.float32),
        pltpu.VMEM((ZROWS, DEG_W), jnp.float32),
        pltpu.VMEM_SHARED((NPAD, DEG_W), jnp.float32),
    ],
)
def _deg_kernel(dst_hbm, out_hbm, didx, ones_v, zbuf, acc):
    c = lax.axis_index("c")
    s = lax.axis_index("s")
    wid = s * NC + c

    _zero_fill(zbuf, ZROWS, DEG_W)

    def row(r, _):
        ones_v[r, pl.ds(0, 16)] = jnp.ones((16,), jnp.float32)
        return 0
    lax.fori_loop(0, CHUNK, row, 0, unroll=False)

    for t in range(RPT // ZROWS):
        pltpu.sync_copy(zbuf, acc.at[pl.ds(s * RPT + t * ZROWS, ZROWS)])
    pltpu.sync_copy(dst_hbm.at[pl.ds(wid * CPT, CPT)], didx)
    plsc.subcore_barrier()

    def body(j, _):
        pltpu.sync_copy(ones_v, acc.at[didx.at[j]], add=True)
        return 0
    lax.fori_loop(0, CPT, body, 0, unroll=False)
    plsc.subcore_barrier()

    pltpu.sync_copy(acc.at[pl.ds(s * RPT, RPT)],
                    out_hbm.at[c, pl.ds(s * RPT, RPT)])


_prop128 = _make_prop(128)
_prop64 = _make_prop(64)

BR = 1000


def _dis_block(d0, d1):
    deg = d0[:, 0:1] + d1[:, 0:1] + 1.0
    return lax.rsqrt(deg)


def _tc1_body(x_ref, d0_ref, d1_ref, w1_ref, u1_ref):
    dis = _dis_block(d0_ref[...], d1_ref[...])
    u1_ref[...] = dis * jnp.dot(x_ref[...], w1_ref[...],
                                preferred_element_type=jnp.float32)


def _tc2_body(a0_ref, a1_ref, u1_ref, d0_ref, d1_ref, b1_ref, w2_ref, u2_ref):
    dis = _dis_block(d0_ref[...], d1_ref[...])
    h1 = jnp.maximum(dis * (a0_ref[...] + a1_ref[...] + u1_ref[...])
                     + b1_ref[...], 0.0)
    u2_ref[...] = dis * jnp.dot(h1, w2_ref[...],
                                preferred_element_type=jnp.float32)


def _tc3_body(a0_ref, a1_ref, u2_ref, d0_ref, d1_ref, b2_ref, u3_ref):
    dis = _dis_block(d0_ref[...], d1_ref[...])
    h2 = dis * (a0_ref[...] + a1_ref[...] + u2_ref[...]) + b2_ref[...]
    u3_ref[...] = dis * h2


def _tc4_body(a0_ref, a1_ref, u3_ref, d0_ref, d1_ref, wmu_ref, bmu_ref,
              wls_ref, bls_ref, eps_ref, z_ref, mu_ref, ls_ref):
    dis = _dis_block(d0_ref[...], d1_ref[...])
    q = dis * (a0_ref[...] + a1_ref[...] + u3_ref[...])
    mu = jnp.dot(q, wmu_ref[...], preferred_element_type=jnp.float32) + bmu_ref[...]
    ls = jnp.dot(q, wls_ref[...], preferred_element_type=jnp.float32) + bls_ref[...]
    mu_ref[...] = mu
    ls_ref[...] = ls
    z_ref[...] = mu + jnp.exp(ls) * eps_ref[...]


def _row_spec(w):
    return pl.BlockSpec((BR, w), lambda i: (i, 0))


def _full_spec(shape):
    return pl.BlockSpec(shape, lambda i: tuple(0 for _ in shape))


def kernel(x, edge_index, W1, b1, W2, b2, Wmu, bmu, Wls, bls):
    src = edge_index[0].astype(jnp.int32)
    dst = edge_index[1].astype(jnp.int32)
    pad = EP - E
    src_p = jnp.concatenate([src, jnp.zeros((pad,), jnp.int32)]).reshape(NCHUNKS, CHUNK)
    dst_p = jnp.concatenate([dst, jnp.full((pad,), N, jnp.int32)]).reshape(NCHUNKS, CHUNK)

    degp = _deg_kernel(dst_p)
    d0, d1 = degp[0], degp[1]

    b1r = b1.reshape(1, HID)
    b2r = b2.reshape(1, LAT)
    bmur = bmu.reshape(1, OUT)
    blsr = bls.reshape(1, OUT)
    eps = jax.random.normal(jax.random.key(42), (N, OUT), dtype=jnp.float32)

    grid = (N // BR,)

    u1 = pl.pallas_call(
        _tc1_body,
        grid=grid,
        in_specs=[_row_spec(D_IN), _row_spec(DEG_W), _row_spec(DEG_W),
                  _full_spec((D_IN, HID))],
        out_specs=_row_spec(HID),
        out_shape=jax.ShapeDtypeStruct((N, HID), jnp.float32),
    )(x, d0, d1, W1)

    a1p = _prop128(src_p, dst_p, u1)

    u2 = pl.pallas_call(
        _tc2_body,
        grid=grid,
        in_specs=[_row_spec(HID), _row_spec(HID), _row_spec(HID),
                  _row_spec(DEG_W), _row_spec(DEG_W),
                  _full_spec((1, HID)), _full_spec((HID, LAT))],
        out_specs=_row_spec(LAT),
        out_shape=jax.ShapeDtypeStruct((N, LAT), jnp.float32),
    )(a1p[0], a1p[1], u1, d0, d1, b1r, W2)

    a2p = _prop64(src_p, dst_p, u2)

    u3 = pl.pallas_call(
        _tc3_body,
        grid=grid,
        in_specs=[_row_spec(LAT), _row_spec(LAT), _row_spec(LAT),
                  _row_spec(DEG_W), _row_spec(DEG_W), _full_spec((1, LAT))],
        out_specs=_row_spec(LAT),
        out_shape=jax.ShapeDtypeStruct((N, LAT), jnp.float32),
    )(a2p[0], a2p[1], u2, d0, d1, b2r)

    a3p = _prop64(src_p, dst_p, u3)

    z, mu, log_std = pl.pallas_call(
        _tc4_body,
        grid=grid,
        in_specs=[_row_spec(LAT), _row_spec(LAT), _row_spec(LAT),
                  _row_spec(DEG_W), _row_spec(DEG_W),
                  _full_spec((LAT, OUT)), _full_spec((1, OUT)),
                  _full_spec((LAT, OUT)), _full_spec((1, OUT)),
                  _row_spec(OUT)],
        out_specs=[_row_spec(OUT), _row_spec(OUT), _row_spec(OUT)],
        out_shape=[jax.ShapeDtypeStruct((N, OUT), jnp.float32)] * 3,
    )(a3p[0], a3p[1], u3, d0, d1, Wmu, bmur, Wls, blsr, eps)

    return (z, mu, log_std)


# speedup vs baseline: 11.9216x; 11.9216x over previous
import functools

import jax
import jax.numpy as jnp
from jax import lax
from jax.experimental import pallas as pl
from jax.experimental.pallas import tpu as pltpu
from jax.experimental.pallas import tpu_sc as plsc

N = 10000
E = 320000
D_IN = 128
HID = 128
LAT = 64
OUT = 32

NC = 2
NS = 16
NW = NC * NS
CHUNK = 128
CPT = 80
HALF = CPT // 2
NCHUNKS = NW * CPT
EP = NCHUNKS * CHUNK
NPAD = 10112
RPT = NPAD // NS

_mesh = plsc.VectorSubcoreMesh(
    core_axis_name="c", subcore_axis_name="s", num_cores=NC, num_subcores=NS)


def _fill(buf, rows, width, value):
    def row(r, _):
        for k in range(width // 16):
            buf[r, pl.ds(k * 16, 16)] = jnp.full((16,), value, jnp.float32)
        return 0
    lax.fori_loop(0, rows, row, 0, unroll=False)


def _zero_acc_slice(zbuf, acc, s):
    for t in range(4):
        pltpu.sync_copy(zbuf, acc.at[pl.ds(s * RPT + t * 128, 128)])
    pltpu.sync_copy(zbuf.at[pl.ds(0, RPT - 512)],
                    acc.at[pl.ds(s * RPT + 512, RPT - 512)])


def _make_prop(width):
    @functools.partial(
        pl.kernel,
        mesh=_mesh,
        compiler_params=pltpu.CompilerParams(use_tc_tiling_on_sc=False),
        out_type=jax.ShapeDtypeStruct((NC, NPAD, width), jnp.float32),
        scratch_types=[
            pltpu.VMEM((HALF, CHUNK), jnp.int32),
            pltpu.VMEM((HALF, CHUNK), jnp.int32),
            pltpu.VMEM((CHUNK, width), jnp.float32),
            pltpu.VMEM((CHUNK, width), jnp.float32),
            pltpu.VMEM_SHARED((NPAD, width), jnp.float32),
            pltpu.SemaphoreType.DMA,
            pltpu.SemaphoreType.DMA,
        ],
    )
    def prop(src_hbm, dst_hbm, u_hbm, out_hbm, sidx, didx, rows_a, rows_b,
             acc, sem_a, sem_b):
        c = lax.axis_index("c")
        s = lax.axis_index("s")
        wid = s * NC + c

        _fill(rows_a, CHUNK, width, 0.0)
        _zero_acc_slice(rows_a, acc, s)
        plsc.subcore_barrier()

        for half in range(2):
            pltpu.sync_copy(src_hbm.at[pl.ds(wid * CPT + half * HALF, HALF)], sidx)
            pltpu.sync_copy(dst_hbm.at[pl.ds(wid * CPT + half * HALF, HALF)], didx)

            pltpu.async_copy(u_hbm.at[sidx.at[0]], rows_a, sem_a)

            def body(i, _):
                j0 = 2 * i
                pltpu.async_copy(u_hbm.at[sidx.at[j0 + 1]], rows_b, sem_b)
                pltpu.make_async_copy(u_hbm.at[sidx.at[j0]], rows_a, sem_a).wait()
                pltpu.sync_copy(rows_a, acc.at[didx.at[j0]], add=True)

                @pl.when(i < HALF // 2 - 1)
                def _():
                    pltpu.async_copy(u_hbm.at[sidx.at[j0 + 2]], rows_a, sem_a)

                pltpu.make_async_copy(u_hbm.at[sidx.at[j0 + 1]], rows_b, sem_b).wait()
                pltpu.sync_copy(rows_b, acc.at[didx.at[j0 + 1]], add=True)
                return 0

            lax.fori_loop(0, HALF // 2, body, 0, unroll=False)
        plsc.subcore_barrier()

        pltpu.sync_copy(acc.at[pl.ds(s * RPT, RPT)],
                        out_hbm.at[c, pl.ds(s * RPT, RPT)])

    return prop


DEG_W = 16


@functools.partial(
    pl.kernel,
    mesh=_mesh,
    compiler_params=pltpu.CompilerParams(use_tc_tiling_on_sc=False),
    out_type=jax.ShapeDtypeStruct((NC, NPAD, DEG_W), jnp.float32),
    scratch_types=[
        pltpu.VMEM((CPT, CHUNK), jnp.int32),
        pltpu.VMEM((CHUNK, DEG_W), jnp.float32),
        pltpu.VMEM_SHARED((NPAD, DEG_W), jnp.float32),
    ],
)
def _deg_kernel(dst_hbm, out_hbm, didx, ones_v, acc):
    c = lax.axis_index("c")
    s = lax.axis_index("s")
    wid = s * NC + c

    _fill(ones_v, CHUNK, DEG_W, 0.0)
    _zero_acc_slice(ones_v, acc, s)
    _fill(ones_v, CHUNK, DEG_W, 1.0)
    pltpu.sync_copy(dst_hbm.at[pl.ds(wid * CPT, CPT)], didx)
    plsc.subcore_barrier()

    def body(j, _):
        pltpu.sync_copy(ones_v, acc.at[didx.at[j]], add=True)
        return 0
    lax.fori_loop(0, CPT, body, 0, unroll=False)
    plsc.subcore_barrier()

    pltpu.sync_copy(acc.at[pl.ds(s * RPT, RPT)],
                    out_hbm.at[c, pl.ds(s * RPT, RPT)])


_prop128 = _make_prop(128)
_prop64 = _make_prop(64)

BR = 1000


def _dis_block(d0, d1):
    deg = d0[:, 0:1] + d1[:, 0:1] + 1.0
    return lax.rsqrt(deg)


def _tc1_body(x_ref, d0_ref, d1_ref, w1_ref, u1_ref):
    dis = _dis_block(d0_ref[...], d1_ref[...])
    u1_ref[...] = dis * jnp.dot(x_ref[...], w1_ref[...],
                                preferred_element_type=jnp.float32)


def _tc2_body(a0_ref, a1_ref, u1_ref, d0_ref, d1_ref, b1_ref, w2_ref, u2_ref):
    dis = _dis_block(d0_ref[...], d1_ref[...])
    h1 = jnp.maximum(dis * (a0_ref[...] + a1_ref[...] + u1_ref[...])
                     + b1_ref[...], 0.0)
    u2_ref[...] = dis * jnp.dot(h1, w2_ref[...],
                                preferred_element_type=jnp.float32)


def _tc3_body(a0_ref, a1_ref, u2_ref, d0_ref, d1_ref, b2_ref, u3_ref):
    dis = _dis_block(d0_ref[...], d1_ref[...])
    h2 = dis * (a0_ref[...] + a1_ref[...] + u2_ref[...]) + b2_ref[...]
    u3_ref[...] = dis * h2


def _tc4_body(a0_ref, a1_ref, u3_ref, d0_ref, d1_ref, wmu_ref, bmu_ref,
              wls_ref, bls_ref, eps_ref, z_ref, mu_ref, ls_ref):
    dis = _dis_block(d0_ref[...], d1_ref[...])
    q = dis * (a0_ref[...] + a1_ref[...] + u3_ref[...])
    mu = jnp.dot(q, wmu_ref[...], preferred_element_type=jnp.float32) + bmu_ref[...]
    ls = jnp.dot(q, wls_ref[...], preferred_element_type=jnp.float32) + bls_ref[...]
    mu_ref[...] = mu
    ls_ref[...] = ls
    z_ref[...] = mu + jnp.exp(ls) * eps_ref[...]


def _row_spec(w):
    return pl.BlockSpec((BR, w), lambda i: (i, 0))


def _full_spec(shape):
    return pl.BlockSpec(shape, lambda i: tuple(0 for _ in shape))


def kernel(x, edge_index, W1, b1, W2, b2, Wmu, bmu, Wls, bls):
    src = edge_index[0].astype(jnp.int32)
    dst = edge_index[1].astype(jnp.int32)
    pad = EP - E
    src_p = jnp.concatenate([src, jnp.zeros((pad,), jnp.int32)]).reshape(NCHUNKS, CHUNK)
    dst_p = jnp.concatenate([dst, jnp.full((pad,), N, jnp.int32)]).reshape(NCHUNKS, CHUNK)

    degp = _deg_kernel(dst_p)
    d0, d1 = degp[0], degp[1]

    b1r = b1.reshape(1, HID)
    b2r = b2.reshape(1, LAT)
    bmur = bmu.reshape(1, OUT)
    blsr = bls.reshape(1, OUT)
    eps = jax.random.normal(jax.random.key(42), (N, OUT), dtype=jnp.float32)

    grid = (N // BR,)

    u1 = pl.pallas_call(
        _tc1_body,
        grid=grid,
        in_specs=[_row_spec(D_IN), _row_spec(DEG_W), _row_spec(DEG_W),
                  _full_spec((D_IN, HID))],
        out_specs=_row_spec(HID),
        out_shape=jax.ShapeDtypeStruct((N, HID), jnp.float32),
    )(x, d0, d1, W1)

    a1p = _prop128(src_p, dst_p, u1)

    u2 = pl.pallas_call(
        _tc2_body,
        grid=grid,
        in_specs=[_row_spec(HID), _row_spec(HID), _row_spec(HID),
                  _row_spec(DEG_W), _row_spec(DEG_W),
                  _full_spec((1, HID)), _full_spec((HID, LAT))],
        out_specs=_row_spec(LAT),
        out_shape=jax.ShapeDtypeStruct((N, LAT), jnp.float32),
    )(a1p[0], a1p[1], u1, d0, d1, b1r, W2)

    a2p = _prop64(src_p, dst_p, u2)

    u3 = pl.pallas_call(
        _tc3_body,
        grid=grid,
        in_specs=[_row_spec(LAT), _row_spec(LAT), _row_spec(LAT),
                  _row_spec(DEG_W), _row_spec(DEG_W), _full_spec((1, LAT))],
        out_specs=_row_spec(LAT),
        out_shape=jax.ShapeDtypeStruct((N, LAT), jnp.float32),
    )(a2p[0], a2p[1], u2, d0, d1, b2r)

    a3p = _prop64(src_p, dst_p, u3)

    z, mu, log_std = pl.pallas_call(
        _tc4_body,
        grid=grid,
        in_specs=[_row_spec(LAT), _row_spec(LAT), _row_spec(LAT),
                  _row_spec(DEG_W), _row_spec(DEG_W),
                  _full_spec((LAT, OUT)), _full_spec((1, OUT)),
                  _full_spec((LAT, OUT)), _full_spec((1, OUT)),
                  _row_spec(OUT)],
        out_specs=[_row_spec(OUT), _row_spec(OUT), _row_spec(OUT)],
        out_shape=[jax.ShapeDtypeStruct((N, OUT), jnp.float32)] * 3,
    )(a3p[0], a3p[1], u3, d0, d1, Wmu, bmur, Wls, blsr, eps)

    return (z, mu, log_std)


# speedup vs baseline: 26.2242x; 2.1997x over previous
import functools

import jax
import jax.numpy as jnp
from jax import lax
from jax.experimental import pallas as pl
from jax.experimental.pallas import tpu as pltpu
from jax.experimental.pallas import tpu_sc as plsc

N = 10000
E = 320000
D_IN = 128
HID = 128
LAT = 64
OUT = 32

NC = 2
NS = 16
NW = NC * NS
CHUNK = 128
CPT = 80
HALF = CPT // 2
NCHUNKS = NW * CPT
EP = NCHUNKS * CHUNK
NPAD = 10112
RPT = NPAD // NS

_mesh = plsc.VectorSubcoreMesh(
    core_axis_name="c", subcore_axis_name="s", num_cores=NC, num_subcores=NS)


def _fill(buf, rows, width, value):
    def row(r, _):
        for k in range(width // 16):
            buf[r, pl.ds(k * 16, 16)] = jnp.full((16,), value, jnp.float32)
        return 0
    lax.fori_loop(0, rows, row, 0, unroll=False)


def _zero_acc_slice(zbuf, acc, s):
    for t in range(4):
        pltpu.sync_copy(zbuf, acc.at[pl.ds(s * RPT + t * 128, 128)])
    pltpu.sync_copy(zbuf.at[pl.ds(0, RPT - 512)],
                    acc.at[pl.ds(s * RPT + 512, RPT - 512)])


TROWS = N // NS


@functools.partial(
    pl.kernel,
    mesh=_mesh,
    compiler_params=pltpu.CompilerParams(use_tc_tiling_on_sc=False),
    out_type=jax.ShapeDtypeStruct((NC, NPAD, LAT), jnp.float32),
    scratch_types=[
        pltpu.VMEM((CPT, CHUNK), jnp.int32),
        pltpu.VMEM((CPT, CHUNK), jnp.int32),
        pltpu.VMEM((CHUNK, LAT), jnp.float32),
        pltpu.VMEM((CHUNK, LAT), jnp.float32),
        pltpu.VMEM_SHARED((N, LAT), jnp.float32),
        pltpu.VMEM_SHARED((NPAD, LAT), jnp.float32),
        pltpu.SemaphoreType.DMA,
        pltpu.SemaphoreType.DMA,
    ],
)
def _prop(src_hbm, dst_hbm, u_hbm, out_hbm, sidx, didx, rows_a, rows_b,
          table, acc, sem_a, sem_b):
    c = lax.axis_index("c")
    s = lax.axis_index("s")
    wid = s * NC + c

    _fill(rows_a, CHUNK, LAT, 0.0)
    _zero_acc_slice(rows_a, acc, s)
    pltpu.sync_copy(u_hbm.at[pl.ds(s * TROWS, TROWS)],
                    table.at[pl.ds(s * TROWS, TROWS)])
    pltpu.sync_copy(src_hbm.at[pl.ds(wid * CPT, CPT)], sidx)
    pltpu.sync_copy(dst_hbm.at[pl.ds(wid * CPT, CPT)], didx)
    plsc.subcore_barrier()

    pltpu.async_copy(table.at[sidx.at[0]], rows_a, sem_a)

    def body(i, _):
        j0 = 2 * i
        pltpu.async_copy(table.at[sidx.at[j0 + 1]], rows_b, sem_b)
        pltpu.make_async_copy(table.at[sidx.at[j0]], rows_a, sem_a).wait()
        pltpu.sync_copy(rows_a, acc.at[didx.at[j0]], add=True)

        @pl.when(i < CPT // 2 - 1)
        def _():
            pltpu.async_copy(table.at[sidx.at[j0 + 2]], rows_a, sem_a)

        pltpu.make_async_copy(table.at[sidx.at[j0 + 1]], rows_b, sem_b).wait()
        pltpu.sync_copy(rows_b, acc.at[didx.at[j0 + 1]], add=True)
        return 0

    lax.fori_loop(0, CPT // 2, body, 0, unroll=False)
    plsc.subcore_barrier()

    pltpu.sync_copy(acc.at[pl.ds(s * RPT, RPT)],
                    out_hbm.at[c, pl.ds(s * RPT, RPT)])


DEG_W = 16


@functools.partial(
    pl.kernel,
    mesh=_mesh,
    compiler_params=pltpu.CompilerParams(use_tc_tiling_on_sc=False),
    out_type=jax.ShapeDtypeStruct((NC, NPAD, DEG_W), jnp.float32),
    scratch_types=[
        pltpu.VMEM((CPT, CHUNK), jnp.int32),
        pltpu.VMEM((CHUNK, DEG_W), jnp.float32),
        pltpu.VMEM_SHARED((NPAD, DEG_W), jnp.float32),
    ],
)
def _deg_kernel(dst_hbm, out_hbm, didx, ones_v, acc):
    c = lax.axis_index("c")
    s = lax.axis_index("s")
    wid = s * NC + c

    _fill(ones_v, CHUNK, DEG_W, 0.0)
    _zero_acc_slice(ones_v, acc, s)
    _fill(ones_v, CHUNK, DEG_W, 1.0)
    pltpu.sync_copy(dst_hbm.at[pl.ds(wid * CPT, CPT)], didx)
    plsc.subcore_barrier()

    def body(j, _):
        pltpu.sync_copy(ones_v, acc.at[didx.at[j]], add=True)
        return 0
    lax.fori_loop(0, CPT, body, 0, unroll=False)
    plsc.subcore_barrier()

    pltpu.sync_copy(acc.at[pl.ds(s * RPT, RPT)],
                    out_hbm.at[c, pl.ds(s * RPT, RPT)])


BR = 1000


def _dis_block(d0, d1):
    deg = d0[:, 0:1] + d1[:, 0:1] + 1.0
    return lax.rsqrt(deg)


def _tc1_body(x_ref, d0_ref, d1_ref, w1_ref, u1a_ref, u1b_ref):
    dis = _dis_block(d0_ref[...], d1_ref[...])
    u1 = dis * jnp.dot(x_ref[...], w1_ref[...],
                       preferred_element_type=jnp.float32)
    u1a_ref[...] = u1[:, :LAT]
    u1b_ref[...] = u1[:, LAT:]


def _tc2_body(aa0_ref, aa1_ref, ab0_ref, ab1_ref, u1a_ref, u1b_ref,
              d0_ref, d1_ref, b1_ref, w2_ref, u2_ref):
    dis = _dis_block(d0_ref[...], d1_ref[...])
    pa = dis * (aa0_ref[...] + aa1_ref[...] + u1a_ref[...])
    pb = dis * (ab0_ref[...] + ab1_ref[...] + u1b_ref[...])
    h1 = jnp.maximum(jnp.concatenate([pa, pb], axis=1) + b1_ref[...], 0.0)
    u2_ref[...] = dis * jnp.dot(h1, w2_ref[...],
                                preferred_element_type=jnp.float32)


def _tc3_body(a0_ref, a1_ref, u2_ref, d0_ref, d1_ref, b2_ref, u3_ref):
    dis = _dis_block(d0_ref[...], d1_ref[...])
    h2 = dis * (a0_ref[...] + a1_ref[...] + u2_ref[...]) + b2_ref[...]
    u3_ref[...] = dis * h2


def _tc4_body(a0_ref, a1_ref, u3_ref, d0_ref, d1_ref, wmu_ref, bmu_ref,
              wls_ref, bls_ref, eps_ref, z_ref, mu_ref, ls_ref):
    dis = _dis_block(d0_ref[...], d1_ref[...])
    q = dis * (a0_ref[...] + a1_ref[...] + u3_ref[...])
    mu = jnp.dot(q, wmu_ref[...], preferred_element_type=jnp.float32) + bmu_ref[...]
    ls = jnp.dot(q, wls_ref[...], preferred_element_type=jnp.float32) + bls_ref[...]
    mu_ref[...] = mu
    ls_ref[...] = ls
    z_ref[...] = mu + jnp.exp(ls) * eps_ref[...]


def _row_spec(w):
    return pl.BlockSpec((BR, w), lambda i: (i, 0))


def _full_spec(shape):
    return pl.BlockSpec(shape, lambda i: tuple(0 for _ in shape))


def kernel(x, edge_index, W1, b1, W2, b2, Wmu, bmu, Wls, bls):
    src = edge_index[0].astype(jnp.int32)
    dst = edge_index[1].astype(jnp.int32)
    pad = EP - E
    src_p = jnp.concatenate([src, jnp.zeros((pad,), jnp.int32)]).reshape(NCHUNKS, CHUNK)
    dst_p = jnp.concatenate([dst, jnp.full((pad,), N, jnp.int32)]).reshape(NCHUNKS, CHUNK)

    degp = _deg_kernel(dst_p)
    d0, d1 = degp[0], degp[1]

    b1r = b1.reshape(1, HID)
    b2r = b2.reshape(1, LAT)
    bmur = bmu.reshape(1, OUT)
    blsr = bls.reshape(1, OUT)
    eps = jax.random.normal(jax.random.key(42), (N, OUT), dtype=jnp.float32)

    grid = (N // BR,)

    u1a, u1b = pl.pallas_call(
        _tc1_body,
        grid=grid,
        in_specs=[_row_spec(D_IN), _row_spec(DEG_W), _row_spec(DEG_W),
                  _full_spec((D_IN, HID))],
        out_specs=[_row_spec(LAT), _row_spec(LAT)],
        out_shape=[jax.ShapeDtypeStruct((N, LAT), jnp.float32)] * 2,
    )(x, d0, d1, W1)

    a1ap = _prop(src_p, dst_p, u1a)
    a1bp = _prop(src_p, dst_p, u1b)

    u2 = pl.pallas_call(
        _tc2_body,
        grid=grid,
        in_specs=[_row_spec(LAT)] * 6 +
                 [_row_spec(DEG_W), _row_spec(DEG_W),
                  _full_spec((1, HID)), _full_spec((HID, LAT))],
        out_specs=_row_spec(LAT),
        out_shape=jax.ShapeDtypeStruct((N, LAT), jnp.float32),
    )(a1ap[0], a1ap[1], a1bp[0], a1bp[1], u1a, u1b, d0, d1, b1r, W2)

    a2p = _prop(src_p, dst_p, u2)

    u3 = pl.pallas_call(
        _tc3_body,
        grid=grid,
        in_specs=[_row_spec(LAT), _row_spec(LAT), _row_spec(LAT),
                  _row_spec(DEG_W), _row_spec(DEG_W), _full_spec((1, LAT))],
        out_specs=_row_spec(LAT),
        out_shape=jax.ShapeDtypeStruct((N, LAT), jnp.float32),
    )(a2p[0], a2p[1], u2, d0, d1, b2r)

    a3p = _prop(src_p, dst_p, u3)

    z, mu, log_std = pl.pallas_call(
        _tc4_body,
        grid=grid,
        in_specs=[_row_spec(LAT), _row_spec(LAT), _row_spec(LAT),
                  _row_spec(DEG_W), _row_spec(DEG_W),
                  _full_spec((LAT, OUT)), _full_spec((1, OUT)),
                  _full_spec((LAT, OUT)), _full_spec((1, OUT)),
                  _row_spec(OUT)],
        out_specs=[_row_spec(OUT), _row_spec(OUT), _row_spec(OUT)],
        out_shape=[jax.ShapeDtypeStruct((N, OUT), jnp.float32)] * 3,
    )(a3p[0], a3p[1], u3, d0, d1, Wmu, bmur, Wls, blsr, eps)

    return (z, mu, log_std)
